# Initial kernel scaffold; baseline (speedup 1.0000x reference)
import jax
import jax.numpy as jnp
from jax.experimental import pallas as pl


def kernel(xyz, emb0, emb1, emb2, emb3):
    raise NotImplementedError("write your pallas kernel here")


# speedup vs baseline: 22.6966x; 22.6966x over previous
import functools

import jax
import jax.numpy as jnp
from jax import lax
from jax.experimental import pallas as pl
from jax.experimental.pallas import tpu as pltpu
from jax.experimental.pallas import tpu_sc as plsc

_SIZE = 2.0
_RES = (16, 32, 64, 128)
_D = 32
_N = 262144
_NW = 32
_B = 128
_PTS = _N // _NW
_NCHUNK = _PTS // _B
_L = 16


def _rsqrt(s):
    i = lax.bitcast_convert_type(s, jnp.int32)
    i = jnp.int32(0x5F3759DF) - lax.shift_right_logical(i, 1)
    y = lax.bitcast_convert_type(i, jnp.float32)
    half = jnp.float32(0.5) * s
    for _ in range(3):
        y = y * (jnp.float32(1.5) - half * y * y)
    return y


def _body(xyz_hbm, t0, t1, t2, t3, out_hbm,
          xs_v, idx_v, wt_v, rows_v, out_v, sem):
    tables = (t0, t1, t2, t3)
    wid = lax.axis_index("s") * 2 + lax.axis_index("c")
    base = wid * _PTS

    pltpu.sync_copy(xyz_hbm.at[:, pl.ds(base, _PTS)], xs_v)

    def chunk_body(c, carry):
        p0 = c * _B
        for lvl in range(4):
            r = _RES[lvl]
            rf = float(r)
            mult = jnp.float32(rf / _SIZE)
            half = jnp.float32(rf / 2.0)

            def idx_body(g, carry2, mult=mult, half=half, r=r, rf=rf):
                off = p0 + g * _L
                x = xs_v[0, pl.ds(off, _L)]
                y = xs_v[1, pl.ds(off, _L)]
                z = xs_v[2, pl.ds(off, _L)]
                nxx = mult * x + half
                nxy = mult * y + half
                nxz = mult * z + half
                iflx = nxx.astype(jnp.int32)
                ifly = nxy.astype(jnp.int32)
                iflz = nxz.astype(jnp.int32)
                flx = iflx.astype(jnp.float32)
                fly = ifly.astype(jnp.float32)
                flz = iflz.astype(jnp.float32)
                cex = jnp.where(flx == nxx, flx, flx + 1.0)
                cey = jnp.where(fly == nxy, fly, fly + 1.0)
                cez = jnp.where(flz == nxz, flz, flz + 1.0)
                ix_b = iflx
                ix_t = cex.astype(jnp.int32)
                iy_b = ifly * r
                iy_t = cey.astype(jnp.int32) * r
                iz_b = iflz * (r * r)
                iz_t = cez.astype(jnp.int32) * (r * r)
                dx_b = flx - nxx
                dx_t = cex - nxx
                dy_b = fly * rf - nxy
                dy_t = cey * rf - nxy
                dz_b = flz * (rf * rf) - nxz
                dz_t = cez * (rf * rf) - nxz
                dists = []
                for k in range(8):
                    xv, ixv = (dx_t, ix_t) if (k & 4) else (dx_b, ix_b)
                    yv, iyv = (dy_t, iy_t) if (k & 2) else (dy_b, iy_b)
                    zv, izv = (dz_t, iz_t) if (k & 1) else (dz_b, iz_b)
                    idx_v[k, pl.ds(g * _L, _L)] = ixv + iyv + izv
                    s = (xv * xv + yv * yv) + zv * zv
                    dists.append(s * _rsqrt(s))
                sod = (((dists[0] + dists[1]) + (dists[2] + dists[3]))
                       + ((dists[4] + dists[5]) + (dists[6] + dists[7])))
                rcp = jnp.float32(1.0) / sod
                for k in range(8):
                    wt_v[k, pl.ds(g * _L, _L)] = dists[k] * rcp
                return carry2

            lax.fori_loop(0, _B // _L, idx_body, 0)
            copies = [
                pltpu.async_copy(tables[lvl].at[idx_v.at[k]], rows_v.at[k], sem)
                for k in range(8)
            ]
            for cp in copies:
                cp.wait()
            col = lvl * _D

            def comb_body(g, carry2, col=col):
                pbase = g * _L
                wvecs = [wt_v[k, pl.ds(pbase, _L)] for k in range(8)]
                for j in range(_L):
                    p = pbase + j
                    a0 = wvecs[0][j] * rows_v[0, p, pl.ds(0, _L)]
                    a1 = wvecs[0][j] * rows_v[0, p, pl.ds(_L, _L)]
                    for k in range(1, 8):
                        wk = wvecs[k][j]
                        a0 = a0 + wk * rows_v[k, p, pl.ds(0, _L)]
                        a1 = a1 + wk * rows_v[k, p, pl.ds(_L, _L)]
                    out_v[p, pl.ds(col, _L)] = a0
                    out_v[p, pl.ds(col + _L, _L)] = a1
                return carry2

            lax.fori_loop(0, _B // _L, comb_body, 0)
        pltpu.sync_copy(out_v, out_hbm.at[pl.ds(base + p0, _B), :])
        return carry

    lax.fori_loop(0, _NCHUNK, chunk_body, 0)


@functools.partial(jax.jit, static_argnums=())
def _run(xyz_t, t0, t1, t2, t3):
    mesh = plsc.VectorSubcoreMesh(core_axis_name="c", subcore_axis_name="s")
    f = pl.kernel(
        _body,
        out_type=jax.ShapeDtypeStruct((_N, 4 * _D), jnp.float32),
        mesh=mesh,
        scratch_types=[
            pltpu.VMEM((3, _PTS), jnp.float32),
            pltpu.VMEM((8, _B), jnp.int32),
            pltpu.VMEM((8, _B), jnp.float32),
            pltpu.VMEM((8, _B, _D), jnp.float32),
            pltpu.VMEM((_B, 4 * _D), jnp.float32),
            pltpu.SemaphoreType.DMA,
        ],
        compiler_params=pltpu.CompilerParams(use_tc_tiling_on_sc=False),
    )
    return f(xyz_t, t0, t1, t2, t3)


def kernel(xyz, emb0, emb1, emb2, emb3):
    xyz_t = xyz.T
    return _run(xyz_t, emb0, emb1, emb2, emb3)


# speedup vs baseline: 26.8454x; 1.1828x over previous
import functools

import jax
import jax.numpy as jnp
from jax import lax
from jax.experimental import pallas as pl
from jax.experimental.pallas import tpu as pltpu
from jax.experimental.pallas import tpu_sc as plsc

_SIZE = 2.0
_RES = (16, 32, 64, 128)
_D = 32
_N = 262144
_NW = 32
_B = 128
_PTS = _N // _NW
_NCHUNK = _PTS // _B
_L = 16


def _rsqrt(s):
    i = lax.bitcast_convert_type(s, jnp.int32)
    i = jnp.int32(0x5F3759DF) - lax.shift_right_logical(i, 1)
    y = lax.bitcast_convert_type(i, jnp.float32)
    half = jnp.float32(0.5) * s
    for _ in range(3):
        y = y * (jnp.float32(1.5) - half * y * y)
    return y


def _body(xyz_hbm, t0, t1, t2, t3, out_hbm,
          xs_v, idx_v, wt_v, rows_v, out_v, gsem0, gsem1, osem):
    tables = (t0, t1, t2, t3)
    gsems = (gsem0, gsem1)
    wid = lax.axis_index("s") * 2 + lax.axis_index("c")
    base = wid * _PTS

    pltpu.sync_copy(xyz_hbm.at[:, pl.ds(base, _PTS)], xs_v)

    def chunk_body(c, carry):
        p0 = c * _B

        def compute_idx(lvl, par):
            r = _RES[lvl]
            rf = float(r)
            mult = jnp.float32(rf / _SIZE)
            half = jnp.float32(rf / 2.0)

            def idx_body(g, carry2, mult=mult, half=half, r=r, rf=rf, par=par):
                off = p0 + g * _L
                x = xs_v[0, pl.ds(off, _L)]
                y = xs_v[1, pl.ds(off, _L)]
                z = xs_v[2, pl.ds(off, _L)]
                nxx = mult * x + half
                nxy = mult * y + half
                nxz = mult * z + half
                iflx = nxx.astype(jnp.int32)
                ifly = nxy.astype(jnp.int32)
                iflz = nxz.astype(jnp.int32)
                flx = iflx.astype(jnp.float32)
                fly = ifly.astype(jnp.float32)
                flz = iflz.astype(jnp.float32)
                cex = jnp.where(flx == nxx, flx, flx + 1.0)
                cey = jnp.where(fly == nxy, fly, fly + 1.0)
                cez = jnp.where(flz == nxz, flz, flz + 1.0)
                ix_b = iflx
                ix_t = cex.astype(jnp.int32)
                iy_b = ifly * r
                iy_t = cey.astype(jnp.int32) * r
                iz_b = iflz * (r * r)
                iz_t = cez.astype(jnp.int32) * (r * r)
                dx_b = flx - nxx
                dx_t = cex - nxx
                dy_b = fly * rf - nxy
                dy_t = cey * rf - nxy
                dz_b = flz * (rf * rf) - nxz
                dz_t = cez * (rf * rf) - nxz
                dists = []
                for k in range(8):
                    xv, ixv = (dx_t, ix_t) if (k & 4) else (dx_b, ix_b)
                    yv, iyv = (dy_t, iy_t) if (k & 2) else (dy_b, iy_b)
                    zv, izv = (dz_t, iz_t) if (k & 1) else (dz_b, iz_b)
                    idx_v[par, k, pl.ds(g * _L, _L)] = ixv + iyv + izv
                    s = (xv * xv + yv * yv) + zv * zv
                    dists.append(s * _rsqrt(s))
                sod = (((dists[0] + dists[1]) + (dists[2] + dists[3]))
                       + ((dists[4] + dists[5]) + (dists[6] + dists[7])))
                rcp = jnp.float32(1.0) / sod
                for k in range(8):
                    wt_v[par, k, pl.ds(g * _L, _L)] = dists[k] * rcp
                return carry2

            lax.fori_loop(0, _B // _L, idx_body, 0)

        def fire(lvl, par):
            return [
                pltpu.async_copy(tables[lvl].at[idx_v.at[par, k]],
                                 rows_v.at[par, k], gsems[par])
                for k in range(8)
            ]

        def combine(lvl, par):
            col = lvl * _D

            def comb_body(g, carry2, col=col, par=par):
                pbase = g * _L
                wvecs = [wt_v[par, k, pl.ds(pbase, _L)] for k in range(8)]
                for j in range(_L):
                    p = pbase + j
                    a0 = wvecs[0][j] * rows_v[par, 0, p, pl.ds(0, _L)]
                    a1 = wvecs[0][j] * rows_v[par, 0, p, pl.ds(_L, _L)]
                    for k in range(1, 8):
                        wk = wvecs[k][j]
                        a0 = a0 + wk * rows_v[par, k, p, pl.ds(0, _L)]
                        a1 = a1 + wk * rows_v[par, k, p, pl.ds(_L, _L)]
                    out_v[p, pl.ds(col, _L)] = a0
                    out_v[p, pl.ds(col + _L, _L)] = a1
                return carry2

            lax.fori_loop(0, _B // _L, comb_body, 0)

        compute_idx(0, 0)
        g0 = fire(0, 0)
        compute_idx(1, 1)
        g1 = fire(1, 1)

        @pl.when(c > 0)
        def _drain_out():
            pltpu.make_async_copy(out_v, out_hbm.at[pl.ds(base, _B), :],
                                  osem).wait()

        for cp in g0:
            cp.wait()
        combine(0, 0)
        compute_idx(2, 0)
        g2 = fire(2, 0)
        for cp in g1:
            cp.wait()
        combine(1, 1)
        compute_idx(3, 1)
        g3 = fire(3, 1)
        for cp in g2:
            cp.wait()
        combine(2, 0)
        for cp in g3:
            cp.wait()
        combine(3, 1)
        pltpu.async_copy(out_v, out_hbm.at[pl.ds(base + p0, _B), :], osem)
        return carry

    lax.fori_loop(0, _NCHUNK, chunk_body, 0)
    pltpu.make_async_copy(out_v, out_hbm.at[pl.ds(base, _B), :], osem).wait()


@functools.partial(jax.jit, static_argnums=())
def _run(xyz_t, t0, t1, t2, t3):
    mesh = plsc.VectorSubcoreMesh(core_axis_name="c", subcore_axis_name="s")
    f = pl.kernel(
        _body,
        out_type=jax.ShapeDtypeStruct((_N, 4 * _D), jnp.float32),
        mesh=mesh,
        scratch_types=[
            pltpu.VMEM((3, _PTS), jnp.float32),
            pltpu.VMEM((2, 8, _B), jnp.int32),
            pltpu.VMEM((2, 8, _B), jnp.float32),
            pltpu.VMEM((2, 8, _B, _D), jnp.float32),
            pltpu.VMEM((_B, 4 * _D), jnp.float32),
            pltpu.SemaphoreType.DMA,
            pltpu.SemaphoreType.DMA,
            pltpu.SemaphoreType.DMA,
        ],
        compiler_params=pltpu.CompilerParams(use_tc_tiling_on_sc=False),
    )
    return f(xyz_t, t0, t1, t2, t3)


def kernel(xyz, emb0, emb1, emb2, emb3):
    xyz_t = xyz.T
    return _run(xyz_t, emb0, emb1, emb2, emb3)
